# baseline (speedup 1.0000x reference)
import functools

import jax
import jax.numpy as jnp
from jax import lax
from jax.experimental import pallas as pl
from jax.experimental.pallas import tpu as pltpu
from jax.experimental.pallas import tpu_sc as plsc

_B = 4
_S = 2048
_D = 1024
_NW = 32
_SROWS = _S // _NW
_CROWS = 32
_CHUNK = _CROWS * _D
_NCHUNK = _SROWS // _CROWS


def _sc_body(x_hbm, emb_hbm, out_hbm, emb_v, x_v):
    wid = lax.axis_index("s") * 2 + lax.axis_index("c")
    s_base = wid * _SROWS * _D
    for c in range(_NCHUNK):
        e_off = s_base + c * _CHUNK
        pltpu.sync_copy(emb_hbm.at[pl.ds(e_off, _CHUNK)], emb_v)
        for b in range(_B):
            x_off = b * _S * _D + e_off
            pltpu.sync_copy(x_hbm.at[pl.ds(x_off, _CHUNK)], x_v)

            def _add(i, _):
                o = i * 16
                x_v[pl.ds(o, 16)] = x_v[pl.ds(o, 16)] + emb_v[pl.ds(o, 16)]
                return 0

            lax.fori_loop(0, _CHUNK // 16, _add, 0)
            pltpu.sync_copy(x_v, out_hbm.at[pl.ds(x_off, _CHUNK)])


@functools.partial(
    pl.kernel,
    mesh=plsc.VectorSubcoreMesh(core_axis_name="c", subcore_axis_name="s"),
    out_type=jax.ShapeDtypeStruct((_B * _S * _D,), jnp.float32),
    scratch_types=[
        pltpu.VMEM((_CHUNK,), jnp.float32),
        pltpu.VMEM((_CHUNK,), jnp.float32),
    ],
)
def _sc_add(x_hbm, emb_hbm, out_hbm, emb_v, x_v):
    _sc_body(x_hbm, emb_hbm, out_hbm, emb_v, x_v)


def kernel(x, emb):
    B, S, D = x.shape
    out = _sc_add(x.reshape(-1), emb[:S].reshape(-1))
    return out.reshape(B, S, D)


# speedup vs baseline: 1.5655x; 1.5655x over previous
import functools

import jax
import jax.numpy as jnp
from jax import lax
from jax.experimental import pallas as pl
from jax.experimental.pallas import tpu as pltpu
from jax.experimental.pallas import tpu_sc as plsc

_B = 4
_S = 2048
_D = 1024
_NW = 32
_SROWS = _S // _NW
_CROWS = 32
_CHUNK = _CROWS * _D
_NCHUNK = _SROWS // _CROWS
_NJOBS = _NCHUNK * _B


def _sc_body(x_hbm, emb_hbm, out_hbm, emb_v, x_v0, x_v1, ld0, ld1, st0, st1):
    wid = lax.axis_index("s") * 2 + lax.axis_index("c")
    s_base = wid * _SROWS * _D
    x_slots = (x_v0, x_v1)
    ld_sems = (ld0, ld1)
    st_sems = (st0, st1)

    def x_off(j):
        c, b = divmod(j, _B)
        return b * _S * _D + s_base + c * _CHUNK

    lds = {}
    sts = {}
    lds[0] = pltpu.async_copy(x_hbm.at[pl.ds(x_off(0), _CHUNK)],
                              x_slots[0], ld_sems[0])
    for j in range(_NJOBS):
        c, b = divmod(j, _B)
        slot = j % 2
        lds[j].wait()
        if b == 0:
            pltpu.sync_copy(emb_hbm.at[pl.ds(s_base + c * _CHUNK, _CHUNK)],
                            emb_v)
        if j + 1 < _NJOBS:
            if j - 1 >= 0:
                sts[j - 1].wait()
            nslot = (j + 1) % 2
            lds[j + 1] = pltpu.async_copy(
                x_hbm.at[pl.ds(x_off(j + 1), _CHUNK)],
                x_slots[nslot], ld_sems[nslot])

        xa = x_slots[slot]

        @plsc.parallel_loop(0, _CHUNK, 16, unroll=8)
        def _add(i):
            xa[pl.ds(i, 16)] = xa[pl.ds(i, 16)] + emb_v[pl.ds(i, 16)]

        sts[j] = pltpu.async_copy(x_slots[slot],
                                  out_hbm.at[pl.ds(x_off(j), _CHUNK)],
                                  st_sems[slot])
    sts[_NJOBS - 2].wait()
    sts[_NJOBS - 1].wait()


@functools.partial(
    pl.kernel,
    mesh=plsc.VectorSubcoreMesh(core_axis_name="c", subcore_axis_name="s"),
    out_type=jax.ShapeDtypeStruct((_B * _S * _D,), jnp.float32),
    scratch_types=[
        pltpu.VMEM((_CHUNK,), jnp.float32),
        pltpu.VMEM((_CHUNK,), jnp.float32),
        pltpu.VMEM((_CHUNK,), jnp.float32),
        pltpu.SemaphoreType.DMA,
        pltpu.SemaphoreType.DMA,
        pltpu.SemaphoreType.DMA,
        pltpu.SemaphoreType.DMA,
    ],
)
def _sc_add(x_hbm, emb_hbm, out_hbm, emb_v, x_v0, x_v1, ld0, ld1, st0, st1):
    _sc_body(x_hbm, emb_hbm, out_hbm, emb_v, x_v0, x_v1, ld0, ld1, st0, st1)


def kernel(x, emb):
    B, S, D = x.shape
    out = _sc_add(x.reshape(-1), emb[:S].reshape(-1))
    return out.reshape(B, S, D)


# speedup vs baseline: 3.7246x; 2.3791x over previous
import functools

import jax
import jax.numpy as jnp
from jax import lax
from jax.experimental import pallas as pl
from jax.experimental.pallas import tpu as pltpu
from jax.experimental.pallas import tpu_sc as plsc

_B = 4
_S = 2048
_D = 1024
_NW = 32
_SROWS = _S // _NW
_CROWS = 32
_NCHUNK = _SROWS // _CROWS
_NJOBS = _NCHUNK * _B
_CHUNK = _CROWS * _D


def _sc_body(x_hbm, emb_hbm, out_hbm, emb_v, x_v0, x_v1, ld0, ld1, st0, st1):
    wid = lax.axis_index("s") * 2 + lax.axis_index("c")
    row0 = wid * _SROWS
    x_slots = (x_v0, x_v1)
    ld_sems = (ld0, ld1)
    st_sems = (st0, st1)

    def rows(j):
        c, b = divmod(j, _B)
        return b, row0 + c * _CROWS

    b0, r0 = rows(0)
    lds = {}
    sts = {}
    lds[0] = pltpu.async_copy(x_hbm.at[b0, pl.ds(r0, _CROWS), :],
                              x_slots[0], ld_sems[0])
    for j in range(_NJOBS):
        c, b = divmod(j, _B)
        slot = j % 2
        lds[j].wait()
        if b == 0:
            pltpu.sync_copy(emb_hbm.at[pl.ds(row0 + c * _CROWS, _CROWS), :],
                            emb_v)
        if j + 1 < _NJOBS:
            if j - 1 >= 0:
                sts[j - 1].wait()
            nb, nr = rows(j + 1)
            nslot = (j + 1) % 2
            lds[j + 1] = pltpu.async_copy(
                x_hbm.at[nb, pl.ds(nr, _CROWS), :],
                x_slots[nslot], ld_sems[nslot])

        xa = x_slots[slot]

        @plsc.parallel_loop(0, _CHUNK, 16, unroll=8)
        def _add(i):
            r = i >> 10
            c16 = pl.multiple_of(i & (_D - 1), 16)
            xa[r, pl.ds(c16, 16)] = (xa[r, pl.ds(c16, 16)]
                                     + emb_v[r, pl.ds(c16, 16)])

        jb, jr = rows(j)
        sts[j] = pltpu.async_copy(x_slots[slot],
                                  out_hbm.at[jb, pl.ds(jr, _CROWS), :],
                                  st_sems[slot])
    sts[_NJOBS - 2].wait()
    sts[_NJOBS - 1].wait()


@functools.partial(
    pl.kernel,
    mesh=plsc.VectorSubcoreMesh(core_axis_name="c", subcore_axis_name="s"),
    out_type=jax.ShapeDtypeStruct((_B, _S, _D), jnp.float32),
    scratch_types=[
        pltpu.VMEM((_CROWS, _D), jnp.float32),
        pltpu.VMEM((_CROWS, _D), jnp.float32),
        pltpu.VMEM((_CROWS, _D), jnp.float32),
        pltpu.SemaphoreType.DMA,
        pltpu.SemaphoreType.DMA,
        pltpu.SemaphoreType.DMA,
        pltpu.SemaphoreType.DMA,
    ],
)
def _sc_add(x_hbm, emb_hbm, out_hbm, emb_v, x_v0, x_v1, ld0, ld1, st0, st1):
    _sc_body(x_hbm, emb_hbm, out_hbm, emb_v, x_v0, x_v1, ld0, ld1, st0, st1)


def kernel(x, emb):
    B, S, D = x.shape
    return _sc_add(x, emb[:S])
